# baseline (speedup 1.0000x reference)
import functools

import jax
import jax.numpy as jnp
from jax import lax
from jax.experimental import pallas as pl
from jax.experimental.pallas import tpu as pltpu
from jax.experimental.pallas import tpu_sc as plsc

NUM_AGENTS = 8
HIDDEN = 1024
NZ = 2048
BATCH = 16384
ROWS = 1024
NBT = BATCH // ROWS

L = 16
SC_COLS = NZ // 2
STRIPE = 128
NSTRIPES = SC_COLS // STRIPE
NQ = 4
Q_ROWS = (NUM_AGENTS * HIDDEN) // NQ
AG_Q = NUM_AGENTS // NQ
CHUNK_ROWS = 128
NCHUNKS = Q_ROWS // CHUNK_ROWS
CHUNKS_PER_AGENT = HIDDEN // CHUNK_ROWS
NBUF = 4
G = STRIPE // L
GQ = (STRIPE // NQ) // L


def _sc_scale_body(b_hbm, w_hbm, aidx_hbm, out_hbm,
                   buf0, buf1, buf2, buf3, sums_ref, part_ref, shared_ref,
                   wv_ref, aidxv_ref, scale_ref, sem0, sem1, sem2, sem3):
    cid = lax.axis_index("c")
    sid = lax.axis_index("s")
    stripe = cid * NQ + sid // NQ
    q = lax.rem(sid, NQ)
    col0 = stripe * STRIPE
    row0 = q * Q_ROWS

    pltpu.sync_copy(w_hbm, wv_ref)
    pltpu.sync_copy(aidx_hbm, aidxv_ref)
    aidx_v = aidxv_ref[...]

    bufs = [buf0, buf1, buf2, buf3]
    sems = [sem0, sem1, sem2, sem3]

    def start(c):
        return pltpu.async_copy(
            b_hbm.at[pl.ds(row0 + c * CHUNK_ROWS, CHUNK_ROWS),
                     pl.ds(col0, STRIPE)],
            bufs[c % NBUF], sems[c % NBUF])

    handles = {}
    for c in range(NBUF):
        handles[c] = start(c)

    for c in range(NCHUNKS):
        handles[c].wait()
        a = c // CHUNKS_PER_AGENT
        buf = bufs[c % NBUF]

        def row_body(i, accs, buf=buf):
            r = i * 4
            t0 = tuple(buf[r, pl.ds(L * g, L)] + buf[r + 1, pl.ds(L * g, L)]
                       for g in range(G))
            t1 = tuple(buf[r + 2, pl.ds(L * g, L)]
                       + buf[r + 3, pl.ds(L * g, L)] for g in range(G))
            return tuple(accs[g] + (t0[g] + t1[g]) for g in range(G))

        accs = lax.fori_loop(
            0, CHUNK_ROWS // 4, row_body,
            tuple(jnp.zeros((L,), jnp.int32) for _ in range(G)))
        for g in range(G):
            if c % CHUNKS_PER_AGENT == 0:
                sums_ref[a, pl.ds(L * g, L)] = accs[g]
            else:
                prev = sums_ref[a, pl.ds(L * g, L)]
                sums_ref[a, pl.ds(L * g, L)] = prev + accs[g]
        if c + NBUF < NCHUNKS:
            handles[c + NBUF] = start(c + NBUF)

    pltpu.sync_copy(sums_ref, shared_ref.at[sid])
    plsc.subcore_barrier()
    quart0 = (sid // NQ) * NQ
    pltpu.sync_copy(shared_ref.at[pl.ds(quart0, NQ)], part_ref)

    def _finalize(qq):
        for gl in range(GQ):
            sl = pl.ds(qq * (GQ * L) + L * gl, L)
            alpha = jnp.zeros((L,), jnp.float32)
            msel = jnp.zeros((L,), jnp.float32)
            for slot in range(NQ):
                for al in range(AG_Q):
                    rel = jnp.minimum(part_ref[slot, al, sl], 1)
                    relf = rel.astype(jnp.float32)
                    alpha = alpha + relf
                    msel = jnp.where(aidx_v == slot * AG_Q + al, relf, msel)
            weights = jnp.zeros((L,), jnp.float32)
            for k in range(NUM_AGENTS + 1):
                wk = wv_ref[pl.ds(L * k, L)]
                weights = jnp.where(alpha == float(k), wk, weights)
            scale_ref[pl.ds(L * gl, L)] = msel * weights

    for qq in range(NQ):
        @pl.when(q == qq)
        def _fin(qq=qq):
            _finalize(qq)

    pltpu.sync_copy(scale_ref,
                    out_hbm.at[pl.ds(col0 + q * (GQ * L), GQ * L)])


_sc_scale = functools.partial(
    pl.kernel,
    out_type=jax.ShapeDtypeStruct((SC_COLS,), jnp.float32),
    mesh=plsc.VectorSubcoreMesh(core_axis_name="c", subcore_axis_name="s"),
    scratch_types=[
        pltpu.VMEM((CHUNK_ROWS, STRIPE), jnp.int32),
        pltpu.VMEM((CHUNK_ROWS, STRIPE), jnp.int32),
        pltpu.VMEM((CHUNK_ROWS, STRIPE), jnp.int32),
        pltpu.VMEM((CHUNK_ROWS, STRIPE), jnp.int32),
        pltpu.VMEM((AG_Q, STRIPE), jnp.int32),
        pltpu.VMEM((NQ, AG_Q, STRIPE), jnp.int32),
        pltpu.VMEM_SHARED((16, AG_Q, STRIPE), jnp.int32),
        pltpu.VMEM(((NUM_AGENTS + 1) * L,), jnp.float32),
        pltpu.VMEM((L,), jnp.int32),
        pltpu.VMEM((GQ * L,), jnp.float32),
        pltpu.SemaphoreType.DMA,
        pltpu.SemaphoreType.DMA,
        pltpu.SemaphoreType.DMA,
        pltpu.SemaphoreType.DMA,
    ],
)(_sc_scale_body)


def _tc_fused_kernel(aidx_ref, b_ref, w_ref, scl_ref, z_ref, out_ref,
                     masks_ref, scale_ref):
    step = pl.program_id(0)
    half = NZ - SC_COLS

    @pl.when(step < NUM_AGENTS)
    def _reduce():
        m = (jnp.max(b_ref[0], axis=0) > 0).astype(jnp.float32)
        masks_ref[pl.ds(step, 1), :] = m[None, :]

        @pl.when(step == NUM_AGENTS - 1)
        def _finalize():
            alpha = jnp.sum(masks_ref[...], axis=0)
            mask_sel = masks_ref[pl.ds(aidx_ref[0], 1), :][0]
            weights = jnp.zeros((half,), jnp.float32)
            for k in range(NUM_AGENTS + 1):
                weights = jnp.where(alpha == float(k), w_ref[0, k], weights)
            scale_ref[0, SC_COLS:] = mask_sel * weights
            scale_ref[0, :SC_COLS] = scl_ref[0, :]

    @pl.when(step >= NUM_AGENTS)
    def _mul():
        out_ref[...] = z_ref[...] * scale_ref[...]


@functools.partial(jax.jit, static_argnames=())
def kernel(Z_hat, B, w, agent_idx):
    w_b = jnp.broadcast_to(w[:, None], (NUM_AGENTS + 1, L)).reshape(-1)
    aidx_b = jnp.full((L,), agent_idx, jnp.int32)
    scale_sc = _sc_scale(B, w_b, aidx_b).reshape(1, SC_COLS)

    B3 = B.reshape(NUM_AGENTS, HIDDEN, NZ)
    w2 = jnp.zeros((1, 16), jnp.float32).at[0, : NUM_AGENTS + 1].set(w)
    aidx = jnp.asarray(agent_idx, jnp.int32).reshape((1,))

    out = pl.pallas_call(
        _tc_fused_kernel,
        grid_spec=pltpu.PrefetchScalarGridSpec(
            num_scalar_prefetch=1,
            grid=(NUM_AGENTS + NBT,),
            in_specs=[
                pl.BlockSpec(
                    (1, HIDDEN, NZ - SC_COLS),
                    lambda s, aidx: (jnp.minimum(s, NUM_AGENTS - 1), 0, 1),
                ),
                pl.BlockSpec((1, 16), lambda s, aidx: (0, 0)),
                pl.BlockSpec((1, SC_COLS), lambda s, aidx: (0, 0)),
                pl.BlockSpec(
                    (ROWS, NZ),
                    lambda s, aidx: (jnp.maximum(s - NUM_AGENTS, 0), 0),
                ),
            ],
            out_specs=pl.BlockSpec(
                (ROWS, NZ),
                lambda s, aidx: (jnp.maximum(s - NUM_AGENTS, 0), 0),
            ),
            scratch_shapes=[
                pltpu.VMEM((NUM_AGENTS, NZ - SC_COLS), jnp.float32),
                pltpu.VMEM((1, NZ), jnp.float32),
            ],
        ),
        out_shape=jax.ShapeDtypeStruct((BATCH, NZ), jnp.float32),
    )(aidx, B3, w2, scale_sc, Z_hat)
    return out


# speedup vs baseline: 1.0467x; 1.0467x over previous
import functools

import jax
import jax.numpy as jnp
from jax import lax
from jax.experimental import pallas as pl
from jax.experimental.pallas import tpu as pltpu
from jax.experimental.pallas import tpu_sc as plsc

NUM_AGENTS = 8
HIDDEN = 1024
NZ = 2048
BATCH = 16384
ROWS = 1024
NBT = BATCH // ROWS

L = 16
SC_COLS = NZ // 2
STRIPE = 128
NSTRIPES = SC_COLS // STRIPE
NQ = 4
Q_ROWS = (NUM_AGENTS * HIDDEN) // NQ
AG_Q = NUM_AGENTS // NQ
CHUNK_ROWS = 128
NCHUNKS = Q_ROWS // CHUNK_ROWS
CHUNKS_PER_AGENT = HIDDEN // CHUNK_ROWS
NBUF = 4
G = STRIPE // L
GQ = (STRIPE // NQ) // L


def _sc_scale_body(b_hbm, w_hbm, aidx_hbm, out_hbm,
                   buf0, buf1, buf2, buf3, sums_ref, part_ref, shared_ref,
                   wv_ref, aidxv_ref, scale_ref, sem0, sem1, sem2, sem3):
    cid = lax.axis_index("c")
    sid = lax.axis_index("s")
    stripe = cid * NQ + sid // NQ
    q = lax.rem(sid, NQ)
    col0 = stripe * STRIPE
    row0 = q * Q_ROWS

    pltpu.sync_copy(w_hbm, wv_ref)
    pltpu.sync_copy(aidx_hbm, aidxv_ref)
    aidx_v = aidxv_ref[...]

    bufs = [buf0, buf1, buf2, buf3]
    sems = [sem0, sem1, sem2, sem3]

    def start(c):
        return pltpu.async_copy(
            b_hbm.at[pl.ds(row0 + c * CHUNK_ROWS, CHUNK_ROWS),
                     pl.ds(col0, STRIPE)],
            bufs[c % NBUF], sems[c % NBUF])

    handles = {}
    for c in range(NBUF):
        handles[c] = start(c)

    for c in range(NCHUNKS):
        handles[c].wait()
        a = c // CHUNKS_PER_AGENT
        buf = bufs[c % NBUF]

        def row_body(i, accs, buf=buf):
            r = i * 4
            t0 = tuple(buf[r, pl.ds(L * g, L)] + buf[r + 1, pl.ds(L * g, L)]
                       for g in range(G))
            t1 = tuple(buf[r + 2, pl.ds(L * g, L)]
                       + buf[r + 3, pl.ds(L * g, L)] for g in range(G))
            return tuple(accs[g] + (t0[g] + t1[g]) for g in range(G))

        accs = lax.fori_loop(
            0, CHUNK_ROWS // 4, row_body,
            tuple(jnp.zeros((L,), jnp.int32) for _ in range(G)))
        for g in range(G):
            if c % CHUNKS_PER_AGENT == 0:
                sums_ref[a, pl.ds(L * g, L)] = accs[g]
            else:
                prev = sums_ref[a, pl.ds(L * g, L)]
                sums_ref[a, pl.ds(L * g, L)] = prev + accs[g]
        if c + NBUF < NCHUNKS:
            handles[c + NBUF] = start(c + NBUF)

    pltpu.sync_copy(sums_ref, shared_ref.at[sid])
    plsc.subcore_barrier()
    quart0 = (sid // NQ) * NQ
    pltpu.sync_copy(shared_ref.at[pl.ds(quart0, NQ)], part_ref)

    def _finalize(qq):
        for gl in range(GQ):
            sl = pl.ds(qq * (GQ * L) + L * gl, L)
            alpha = jnp.zeros((L,), jnp.float32)
            msel = jnp.zeros((L,), jnp.float32)
            for slot in range(NQ):
                for al in range(AG_Q):
                    rel = jnp.minimum(part_ref[slot, al, sl], 1)
                    relf = rel.astype(jnp.float32)
                    alpha = alpha + relf
                    msel = jnp.where(aidx_v == slot * AG_Q + al, relf, msel)
            weights = jnp.zeros((L,), jnp.float32)
            for k in range(NUM_AGENTS + 1):
                wk = wv_ref[pl.ds(L * k, L)]
                weights = jnp.where(alpha == float(k), wk, weights)
            scale_ref[pl.ds(L * gl, L)] = msel * weights

    for qq in range(NQ):
        @pl.when(q == qq)
        def _fin(qq=qq):
            _finalize(qq)

    pltpu.sync_copy(scale_ref,
                    out_hbm.at[pl.ds(col0 + q * (GQ * L), GQ * L)])


_sc_scale = functools.partial(
    pl.kernel,
    out_type=jax.ShapeDtypeStruct((SC_COLS,), jnp.float32),
    mesh=plsc.VectorSubcoreMesh(core_axis_name="c", subcore_axis_name="s"),
    scratch_types=[
        pltpu.VMEM((CHUNK_ROWS, STRIPE), jnp.int32),
        pltpu.VMEM((CHUNK_ROWS, STRIPE), jnp.int32),
        pltpu.VMEM((CHUNK_ROWS, STRIPE), jnp.int32),
        pltpu.VMEM((CHUNK_ROWS, STRIPE), jnp.int32),
        pltpu.VMEM((AG_Q, STRIPE), jnp.int32),
        pltpu.VMEM((NQ, AG_Q, STRIPE), jnp.int32),
        pltpu.VMEM_SHARED((16, AG_Q, STRIPE), jnp.int32),
        pltpu.VMEM(((NUM_AGENTS + 1) * L,), jnp.float32),
        pltpu.VMEM((L,), jnp.int32),
        pltpu.VMEM((GQ * L,), jnp.float32),
        pltpu.SemaphoreType.DMA,
        pltpu.SemaphoreType.DMA,
        pltpu.SemaphoreType.DMA,
        pltpu.SemaphoreType.DMA,
    ],
)(_sc_scale_body)


def _tc_reduce_kernel(aidx_ref, b_ref, w_ref, out_ref, masks_ref):
    a = pl.program_id(0)
    half = NZ - SC_COLS
    m = (jnp.max(b_ref[0], axis=0) > 0).astype(jnp.float32)
    masks_ref[pl.ds(a, 1), :] = m[None, :]

    @pl.when(a == NUM_AGENTS - 1)
    def _finalize():
        alpha = jnp.sum(masks_ref[...], axis=0)
        mask_sel = masks_ref[pl.ds(aidx_ref[0], 1), :][0]
        weights = jnp.zeros((half,), jnp.float32)
        for k in range(NUM_AGENTS + 1):
            weights = jnp.where(alpha == float(k), w_ref[0, k], weights)
        out_ref[0, :] = mask_sel * weights


def _tc_mul_kernel(z_ref, sl_ref, sr_ref, out_ref):
    out_ref[:, :SC_COLS] = z_ref[:, :SC_COLS] * sl_ref[...]
    out_ref[:, SC_COLS:] = z_ref[:, SC_COLS:] * sr_ref[...]


@functools.partial(jax.jit, static_argnames=())
def kernel(Z_hat, B, w, agent_idx):
    w_b = jnp.broadcast_to(w[:, None], (NUM_AGENTS + 1, L)).reshape(-1)
    aidx_b = jnp.full((L,), agent_idx, jnp.int32)
    scale_sc = _sc_scale(B, w_b, aidx_b).reshape(1, SC_COLS)

    B3 = B.reshape(NUM_AGENTS, HIDDEN, NZ)
    w2 = jnp.zeros((1, 16), jnp.float32).at[0, : NUM_AGENTS + 1].set(w)
    aidx = jnp.asarray(agent_idx, jnp.int32).reshape((1,))

    scale_tc = pl.pallas_call(
        _tc_reduce_kernel,
        grid_spec=pltpu.PrefetchScalarGridSpec(
            num_scalar_prefetch=1,
            grid=(NUM_AGENTS,),
            in_specs=[
                pl.BlockSpec((1, HIDDEN, NZ - SC_COLS),
                             lambda a, aidx: (a, 0, 1)),
                pl.BlockSpec((1, 16), lambda a, aidx: (0, 0)),
            ],
            out_specs=pl.BlockSpec((1, NZ - SC_COLS),
                                   lambda a, aidx: (0, 0)),
            scratch_shapes=[
                pltpu.VMEM((NUM_AGENTS, NZ - SC_COLS), jnp.float32),
            ],
        ),
        out_shape=jax.ShapeDtypeStruct((1, NZ - SC_COLS), jnp.float32),
    )(aidx, B3, w2)

    out = pl.pallas_call(
        _tc_mul_kernel,
        grid=(NBT,),
        in_specs=[
            pl.BlockSpec((ROWS, NZ), lambda i: (i, 0)),
            pl.BlockSpec((1, SC_COLS), lambda i: (0, 0)),
            pl.BlockSpec((1, NZ - SC_COLS), lambda i: (0, 0)),
        ],
        out_specs=pl.BlockSpec((ROWS, NZ), lambda i: (i, 0)),
        out_shape=jax.ShapeDtypeStruct((BATCH, NZ), jnp.float32),
    )(Z_hat, scale_sc, scale_tc)
    return out


# speedup vs baseline: 1.0931x; 1.0444x over previous
import functools

import jax
import jax.numpy as jnp
from jax import lax
from jax.experimental import pallas as pl
from jax.experimental.pallas import tpu as pltpu
from jax.experimental.pallas import tpu_sc as plsc

NUM_AGENTS = 8
HIDDEN = 1024
NZ = 2048
BATCH = 16384
ROWS = 1024
NBT = BATCH // ROWS
SC_AGENTS = 2
TC_AGENTS = NUM_AGENTS - SC_AGENTS

L = 16
STRIPE = 128
NSTRIPES = NZ // STRIPE
CHUNK_ROWS = 128
NCHUNKS = HIDDEN // CHUNK_ROWS
NBUF = 4
G = STRIPE // L
GH = G // 2


def _sc_reduce_body(b_hbm, aidx_hbm, alpha_hbm, msel_hbm,
                    buf0, buf1, buf2, buf3, sums_ref, part_ref, shared_ref,
                    aidxv_ref, av_ref, mv_ref, sem0, sem1, sem2, sem3):
    cid = lax.axis_index("c")
    sid = lax.axis_index("s")
    stripe = cid * (NSTRIPES // 2) + lax.rem(sid, NSTRIPES // 2)
    h = sid // (NSTRIPES // 2)
    col0 = stripe * STRIPE
    row0 = h * HIDDEN

    pltpu.sync_copy(aidx_hbm, aidxv_ref)
    aidx_v = aidxv_ref[...]

    bufs = [buf0, buf1, buf2, buf3]
    sems = [sem0, sem1, sem2, sem3]

    def start(c):
        return pltpu.async_copy(
            b_hbm.at[pl.ds(row0 + c * CHUNK_ROWS, CHUNK_ROWS),
                     pl.ds(col0, STRIPE)],
            bufs[c % NBUF], sems[c % NBUF])

    handles = {}
    for c in range(NBUF):
        handles[c] = start(c)

    for c in range(NCHUNKS):
        handles[c].wait()
        buf = bufs[c % NBUF]

        def row_body(i, accs, buf=buf):
            r = i * 8
            out = []
            for g in range(G):
                t0 = buf[r, pl.ds(L * g, L)] + buf[r + 1, pl.ds(L * g, L)]
                t1 = buf[r + 2, pl.ds(L * g, L)] + buf[r + 3, pl.ds(L * g, L)]
                t2 = buf[r + 4, pl.ds(L * g, L)] + buf[r + 5, pl.ds(L * g, L)]
                t3 = buf[r + 6, pl.ds(L * g, L)] + buf[r + 7, pl.ds(L * g, L)]
                out.append(accs[g] + ((t0 + t1) + (t2 + t3)))
            return tuple(out)

        accs = lax.fori_loop(
            0, CHUNK_ROWS // 8, row_body,
            tuple(jnp.zeros((L,), jnp.int32) for _ in range(G)))
        for g in range(G):
            if c == 0:
                sums_ref[0, pl.ds(L * g, L)] = accs[g]
            else:
                prev = sums_ref[0, pl.ds(L * g, L)]
                sums_ref[0, pl.ds(L * g, L)] = prev + accs[g]
        if c + NBUF < NCHUNKS:
            handles[c + NBUF] = start(c + NBUF)

    pltpu.sync_copy(sums_ref, shared_ref.at[sid])
    plsc.subcore_barrier()
    partner = lax.rem(sid + NSTRIPES // 2, NSTRIPES)
    pltpu.sync_copy(shared_ref.at[partner], part_ref)

    def _finalize(hh):
        for gl in range(GH):
            sl = pl.ds(hh * (GH * L) + L * gl, L)
            relm = jnp.minimum(sums_ref[0, sl], 1).astype(jnp.float32)
            relp = jnp.minimum(part_ref[0, sl], 1).astype(jnp.float32)
            zero = jnp.zeros((L,), jnp.float32)
            msel = (jnp.where(aidx_v == hh, relm, zero)
                    + jnp.where(aidx_v == 1 - hh, relp, zero))
            av_ref[pl.ds(L * gl, L)] = relm + relp
            mv_ref[pl.ds(L * gl, L)] = msel

    @pl.when(h == 0)
    def _h0():
        _finalize(0)

    @pl.when(h == 1)
    def _h1():
        _finalize(1)

    out_off = col0 + h * (GH * L)
    pltpu.sync_copy(av_ref, alpha_hbm.at[pl.ds(out_off, GH * L)])
    pltpu.sync_copy(mv_ref, msel_hbm.at[pl.ds(out_off, GH * L)])


_sc_reduce = functools.partial(
    pl.kernel,
    out_type=(jax.ShapeDtypeStruct((NZ,), jnp.float32),
              jax.ShapeDtypeStruct((NZ,), jnp.float32)),
    mesh=plsc.VectorSubcoreMesh(core_axis_name="c", subcore_axis_name="s"),
    scratch_types=[
        pltpu.VMEM((CHUNK_ROWS, STRIPE), jnp.int32),
        pltpu.VMEM((CHUNK_ROWS, STRIPE), jnp.int32),
        pltpu.VMEM((CHUNK_ROWS, STRIPE), jnp.int32),
        pltpu.VMEM((CHUNK_ROWS, STRIPE), jnp.int32),
        pltpu.VMEM((1, STRIPE), jnp.int32),
        pltpu.VMEM((1, STRIPE), jnp.int32),
        pltpu.VMEM_SHARED((16, 1, STRIPE), jnp.int32),
        pltpu.VMEM((L,), jnp.int32),
        pltpu.VMEM((GH * L,), jnp.float32),
        pltpu.VMEM((GH * L,), jnp.float32),
        pltpu.SemaphoreType.DMA,
        pltpu.SemaphoreType.DMA,
        pltpu.SemaphoreType.DMA,
        pltpu.SemaphoreType.DMA,
    ],
)(_sc_reduce_body)


def _tc_reduce_kernel(aidx_ref, b_ref, alpha_ref, msel_ref):
    a = pl.program_id(0)
    m = (jnp.max(b_ref[0], axis=0) > 0).astype(jnp.float32)
    msk = jnp.where(aidx_ref[0] == a + SC_AGENTS, m, 0.0)

    @pl.when(a == 0)
    def _init():
        alpha_ref[0, :] = m
        msel_ref[0, :] = msk

    @pl.when(a > 0)
    def _acc():
        alpha_ref[0, :] += m
        msel_ref[0, :] += msk


def _tc_mul_kernel(aidx_ref, w_ref, a01_ref, m01_ref, a27_ref, m27_ref,
                   z_ref, out_ref, scale_ref):
    step = pl.program_id(0)

    @pl.when(step == 0)
    def _join():
        alpha = a01_ref[0, :] + a27_ref[0, :]
        msel = m01_ref[0, :] + m27_ref[0, :]
        weights = jnp.zeros((NZ,), jnp.float32)
        for k in range(NUM_AGENTS + 1):
            weights = jnp.where(alpha == float(k), w_ref[0, k], weights)
        scale_ref[0, :] = msel * weights

    out_ref[...] = z_ref[...] * scale_ref[...]


@functools.partial(jax.jit, static_argnames=())
def kernel(Z_hat, B, w, agent_idx):
    aidx_b = jnp.full((L,), agent_idx, jnp.int32)
    alpha01, msel01 = _sc_reduce(B, aidx_b)
    alpha01 = alpha01.reshape(1, NZ)
    msel01 = msel01.reshape(1, NZ)

    B3 = B.reshape(NUM_AGENTS, HIDDEN, NZ)
    w2 = jnp.zeros((1, 16), jnp.float32).at[0, : NUM_AGENTS + 1].set(w)
    aidx = jnp.asarray(agent_idx, jnp.int32).reshape((1,))

    alpha27, msel27 = pl.pallas_call(
        _tc_reduce_kernel,
        grid_spec=pltpu.PrefetchScalarGridSpec(
            num_scalar_prefetch=1,
            grid=(TC_AGENTS,),
            in_specs=[
                pl.BlockSpec((1, HIDDEN, NZ),
                             lambda a, aidx: (a + SC_AGENTS, 0, 0)),
            ],
            out_specs=[
                pl.BlockSpec((1, NZ), lambda a, aidx: (0, 0)),
                pl.BlockSpec((1, NZ), lambda a, aidx: (0, 0)),
            ],
        ),
        out_shape=[
            jax.ShapeDtypeStruct((1, NZ), jnp.float32),
            jax.ShapeDtypeStruct((1, NZ), jnp.float32),
        ],
    )(aidx, B3)

    out = pl.pallas_call(
        _tc_mul_kernel,
        grid_spec=pltpu.PrefetchScalarGridSpec(
            num_scalar_prefetch=1,
            grid=(NBT,),
            in_specs=[
                pl.BlockSpec((1, 16), lambda i, aidx: (0, 0)),
                pl.BlockSpec((1, NZ), lambda i, aidx: (0, 0)),
                pl.BlockSpec((1, NZ), lambda i, aidx: (0, 0)),
                pl.BlockSpec((1, NZ), lambda i, aidx: (0, 0)),
                pl.BlockSpec((1, NZ), lambda i, aidx: (0, 0)),
                pl.BlockSpec((ROWS, NZ), lambda i, aidx: (i, 0)),
            ],
            out_specs=pl.BlockSpec((ROWS, NZ), lambda i, aidx: (i, 0)),
            scratch_shapes=[
                pltpu.VMEM((1, NZ), jnp.float32),
            ],
        ),
        out_shape=jax.ShapeDtypeStruct((BATCH, NZ), jnp.float32),
    )(aidx, w2, alpha01, msel01, alpha27, msel27, Z_hat)
    return out


# speedup vs baseline: 1.2319x; 1.1270x over previous
import functools

import jax
import jax.numpy as jnp
from jax.experimental import pallas as pl
from jax.experimental.pallas import tpu as pltpu

NUM_AGENTS = 8
HIDDEN = 1024
NZ = 2048
BATCH = 16384
ROWS = 512


def _scale_kernel(aidx_ref, b_ref, w_ref, out_ref, masks_ref):
    a = pl.program_id(0)
    m = (jnp.max(b_ref[0], axis=0) > 0).astype(jnp.float32)
    masks_ref[a, :] = m

    @pl.when(a == NUM_AGENTS - 1)
    def _finalize():
        alpha = jnp.sum(masks_ref[...], axis=0)
        aidx = aidx_ref[0]
        mask_sel = masks_ref[pl.ds(aidx, 1), :][0]
        weights = jnp.zeros((NZ,), jnp.float32)
        for k in range(NUM_AGENTS + 1):
            weights = jnp.where(alpha == float(k), w_ref[0, k], weights)
        out_ref[0, :] = mask_sel * weights


def _mul_kernel(z_ref, s_ref, out_ref):
    out_ref[...] = z_ref[...] * s_ref[...]


@functools.partial(jax.jit, static_argnames=())
def kernel(Z_hat, B, w, agent_idx):
    B3 = B.reshape(NUM_AGENTS, HIDDEN, NZ)
    w2 = jnp.zeros((1, 16), jnp.float32).at[0, : NUM_AGENTS + 1].set(w)
    aidx = jnp.asarray(agent_idx, jnp.int32).reshape((1,))

    scale = pl.pallas_call(
        _scale_kernel,
        grid_spec=pltpu.PrefetchScalarGridSpec(
            num_scalar_prefetch=1,
            grid=(NUM_AGENTS,),
            in_specs=[
                pl.BlockSpec((1, HIDDEN, NZ), lambda a, aidx: (a, 0, 0)),
                pl.BlockSpec((1, 16), lambda a, aidx: (0, 0)),
            ],
            out_specs=pl.BlockSpec((1, NZ), lambda a, aidx: (0, 0)),
            scratch_shapes=[pltpu.VMEM((NUM_AGENTS, NZ), jnp.float32)],
        ),
        out_shape=jax.ShapeDtypeStruct((1, NZ), jnp.float32),
    )(aidx, B3, w2)

    out = pl.pallas_call(
        _mul_kernel,
        grid=(BATCH // ROWS,),
        in_specs=[
            pl.BlockSpec((ROWS, NZ), lambda i: (i, 0)),
            pl.BlockSpec((1, NZ), lambda i: (0, 0)),
        ],
        out_specs=pl.BlockSpec((ROWS, NZ), lambda i: (i, 0)),
        out_shape=jax.ShapeDtypeStruct((BATCH, NZ), jnp.float32),
    )(Z_hat, scale)
    return out


# speedup vs baseline: 1.2531x; 1.0172x over previous
import functools

import jax
import jax.numpy as jnp
from jax.experimental import pallas as pl
from jax.experimental.pallas import tpu as pltpu

NUM_AGENTS = 8
HIDDEN = 1024
NZ = 2048
BATCH = 16384
ROWS = 1024


def _scale_kernel(aidx_ref, b_ref, w_ref, out_ref, masks_ref):
    a = pl.program_id(0)
    m = (jnp.max(b_ref[0], axis=0) > 0).astype(jnp.float32)
    masks_ref[a, :] = m

    @pl.when(a == NUM_AGENTS - 1)
    def _finalize():
        alpha = jnp.sum(masks_ref[...], axis=0)
        aidx = aidx_ref[0]
        mask_sel = masks_ref[pl.ds(aidx, 1), :][0]
        weights = jnp.zeros((NZ,), jnp.float32)
        for k in range(NUM_AGENTS + 1):
            weights = jnp.where(alpha == float(k), w_ref[0, k], weights)
        out_ref[0, :] = mask_sel * weights


HALF = NZ // 2


def _mul_kernel(z_ref, s_ref, out_ref):
    out_ref[:, :HALF] = z_ref[:, :HALF] * s_ref[:, :HALF]
    out_ref[:, HALF:] = z_ref[:, HALF:] * s_ref[:, HALF:]


@functools.partial(jax.jit, static_argnames=())
def kernel(Z_hat, B, w, agent_idx):
    B3 = B.reshape(NUM_AGENTS, HIDDEN, NZ)
    w2 = jnp.zeros((1, 16), jnp.float32).at[0, : NUM_AGENTS + 1].set(w)
    aidx = jnp.asarray(agent_idx, jnp.int32).reshape((1,))

    scale = pl.pallas_call(
        _scale_kernel,
        grid_spec=pltpu.PrefetchScalarGridSpec(
            num_scalar_prefetch=1,
            grid=(NUM_AGENTS,),
            in_specs=[
                pl.BlockSpec((1, HIDDEN, NZ), lambda a, aidx: (a, 0, 0)),
                pl.BlockSpec((1, 16), lambda a, aidx: (0, 0)),
            ],
            out_specs=pl.BlockSpec((1, NZ), lambda a, aidx: (0, 0)),
            scratch_shapes=[pltpu.VMEM((NUM_AGENTS, NZ), jnp.float32)],
        ),
        out_shape=jax.ShapeDtypeStruct((1, NZ), jnp.float32),
    )(aidx, B3, w2)

    out = pl.pallas_call(
        _mul_kernel,
        grid=(BATCH // ROWS,),
        in_specs=[
            pl.BlockSpec((ROWS, NZ), lambda i: (i, 0)),
            pl.BlockSpec((1, NZ), lambda i: (0, 0)),
        ],
        out_specs=pl.BlockSpec((ROWS, NZ), lambda i: (i, 0)),
        out_shape=jax.ShapeDtypeStruct((BATCH, NZ), jnp.float32),
    )(Z_hat, scale)
    return out


# speedup vs baseline: 1.2609x; 1.0062x over previous
import functools

import jax
import jax.numpy as jnp
from jax.experimental import pallas as pl
from jax.experimental.pallas import tpu as pltpu

NUM_AGENTS = 8
HIDDEN = 1024
NZ = 2048
BATCH = 16384
ROWS = 1024
RB = 512
NSPLIT = HIDDEN // RB
NBSTEPS = NUM_AGENTS * NSPLIT


def _scale_kernel(aidx_ref, b_ref, w_ref, out_ref, parts_ref):
    i = pl.program_id(0)
    parts_ref[i, :] = jnp.max(b_ref[0], axis=0).astype(jnp.float32)

    @pl.when(i == NBSTEPS - 1)
    def _finalize():
        parts = parts_ref[...].reshape(NUM_AGENTS, NSPLIT, NZ)
        masks = (jnp.max(parts, axis=1) > 0).astype(jnp.float32)
        alpha = jnp.sum(masks, axis=0)
        aidx = aidx_ref[0]
        onehot = (jax.lax.broadcasted_iota(jnp.int32, (NUM_AGENTS, 1), 0)
                  == aidx).astype(jnp.float32)
        mask_sel = jnp.sum(masks * onehot, axis=0)
        weights = jnp.zeros((NZ,), jnp.float32)
        for k in range(NUM_AGENTS + 1):
            weights = jnp.where(alpha == float(k), w_ref[0, k], weights)
        out_ref[0, :] = mask_sel * weights


def _mul_kernel(z_ref, s_ref, out_ref):
    out_ref[...] = z_ref[...] * s_ref[...]


@functools.partial(jax.jit, static_argnames=())
def kernel(Z_hat, B, w, agent_idx):
    B3 = B.reshape(NBSTEPS, RB, NZ)
    w2 = w.reshape(1, NUM_AGENTS + 1)
    aidx = jnp.asarray(agent_idx, jnp.int32).reshape((1,))

    scale = pl.pallas_call(
        _scale_kernel,
        grid_spec=pltpu.PrefetchScalarGridSpec(
            num_scalar_prefetch=1,
            grid=(NBSTEPS,),
            in_specs=[
                pl.BlockSpec((1, RB, NZ), lambda i, aidx: (i, 0, 0)),
                pl.BlockSpec((1, NUM_AGENTS + 1), lambda i, aidx: (0, 0)),
            ],
            out_specs=pl.BlockSpec((1, NZ), lambda i, aidx: (0, 0)),
            scratch_shapes=[pltpu.VMEM((NBSTEPS, NZ), jnp.float32)],
        ),
        out_shape=jax.ShapeDtypeStruct((1, NZ), jnp.float32),
    )(aidx, B3, w2)

    out = pl.pallas_call(
        _mul_kernel,
        grid=(BATCH // ROWS,),
        in_specs=[
            pl.BlockSpec((ROWS, NZ), lambda i: (i, 0)),
            pl.BlockSpec((1, NZ), lambda i: (0, 0)),
        ],
        out_specs=pl.BlockSpec((ROWS, NZ), lambda i: (i, 0)),
        out_shape=jax.ShapeDtypeStruct((BATCH, NZ), jnp.float32),
    )(Z_hat, scale)
    return out
